# Initial kernel scaffold; baseline (speedup 1.0000x reference)
import jax
import jax.numpy as jnp
from jax.experimental import pallas as pl


def kernel(features, edge_index, W):
    raise NotImplementedError("write your pallas kernel here")


# speedup vs baseline: 4.2855x; 4.2855x over previous
import functools

import jax
import jax.numpy as jnp
from jax import lax
from jax.experimental import pallas as pl
from jax.experimental.pallas import tpu as pltpu
from jax.experimental.pallas import tpu_sc as plsc

NC = 2
NS = 16
L = 16
NW = NC * NS

CH = 128


def _sc_scatter(n_nodes, d, ep, acc_rows):
    e_per_tile = ep // NW
    n_ch = e_per_tile // CH
    rows_per_tile = acc_rows // NS
    n_zero = rows_per_tile // CH

    mesh = plsc.VectorSubcoreMesh(core_axis_name="c", subcore_axis_name="s")

    @functools.partial(
        pl.kernel,
        mesh=mesh,
        out_type=jax.ShapeDtypeStruct((NC, acc_rows, d), jnp.float32),
        scratch_types=[
            pltpu.VMEM((CH,), jnp.int32),
            pltpu.VMEM((CH,), jnp.int32),
            pltpu.VMEM((CH, d), jnp.float32),
            pltpu.VMEM_SHARED((acc_rows, d), jnp.float32),
        ],
    )
    def k(feat_hbm, src_hbm, dst_hbm, out_hbm, src_v, dst_v, rows_v, acc_sh):
        cid = lax.axis_index("c")
        sid = lax.axis_index("s")
        wid = sid * NC + cid

        def zero_body(i, _):
            rows_v[i // (d // L), pl.ds((i % (d // L)) * L, L)] = jnp.zeros(
                (L,), jnp.float32)
            return _
        lax.fori_loop(0, CH * (d // L), zero_body, None)
        acc_base = sid * rows_per_tile
        for j in range(n_zero):
            pltpu.sync_copy(rows_v, acc_sh.at[pl.ds(acc_base + j * CH, CH)])
        plsc.subcore_barrier()

        ebase = wid * e_per_tile

        def edge_body(c, _):
            off = ebase + c * CH
            pltpu.sync_copy(src_hbm.at[pl.ds(off, CH)], src_v)
            pltpu.sync_copy(dst_hbm.at[pl.ds(off, CH)], dst_v)
            pltpu.sync_copy(feat_hbm.at[src_v], rows_v)
            pltpu.sync_copy(rows_v, acc_sh.at[dst_v], add=True)
            return _
        lax.fori_loop(0, n_ch, edge_body, None)
        plsc.subcore_barrier()

        pltpu.sync_copy(
            acc_sh.at[pl.ds(acc_base, rows_per_tile)],
            out_hbm.at[cid, pl.ds(acc_base, rows_per_tile)],
        )

    return k


def _combine_body(p0_ref, p1_ref, w_ref, o_ref):
    o_ref[...] = (p0_ref[0] + p1_ref[0]) * w_ref[...]


def kernel(features, edge_index, W):
    n_nodes, d = features.shape
    e = edge_index.shape[1]

    ep = -(-e // (NW * CH)) * (NW * CH)
    src = edge_index[0]
    dst = edge_index[1]
    if ep != e:
        pad = ep - e
        src = jnp.concatenate([src, jnp.zeros((pad,), jnp.int32)])
        dst = jnp.concatenate([dst, jnp.full((pad,), n_nodes, jnp.int32)])

    acc_rows = -(-(n_nodes + 1) // (NS * CH)) * (NS * CH)
    partial = _sc_scatter(n_nodes, d, ep, acc_rows)(features, src, dst)

    blk = 1000
    grid = n_nodes // blk
    out = pl.pallas_call(
        _combine_body,
        grid=(grid,),
        in_specs=[
            pl.BlockSpec((1, blk, d), lambda i: (0, i, 0)),
            pl.BlockSpec((1, blk, d), lambda i: (1, i, 0)),
            pl.BlockSpec((1, d), lambda i: (0, 0)),
        ],
        out_specs=pl.BlockSpec((blk, d), lambda i: (i, 0)),
        out_shape=jax.ShapeDtypeStruct((n_nodes, d), jnp.float32),
    )(partial, partial, W.reshape(1, d))
    return out
